# baseline (speedup 1.0000x reference)
import functools

import jax
import jax.numpy as jnp
from jax import lax
from jax.experimental import pallas as pl
from jax.experimental.pallas import tpu as pltpu
from jax.experimental.pallas import tpu_sc as plsc

_N = 10000
_E = 320000
_D = 128
_H = 64
_G = 64
_EPS = 1e-5

_NC = 2
_NS = 16
_NW = _NC * _NS
_CH = 128
_K = 80
_TILE_E = _K * _CH
_EP = _NW * _TILE_E
_NP = 10240
_RPS = _NP // _NS
_BR = 1024
_NB = _NP // _BR


def _make_sc_degree():
    mesh = plsc.VectorSubcoreMesh(core_axis_name="c", subcore_axis_name="s")

    @functools.partial(
        pl.kernel,
        out_type=jax.ShapeDtypeStruct((_NC * _NP, 16), jnp.float32),
        mesh=mesh,
        compiler_params=pltpu.CompilerParams(use_tc_tiling_on_sc=False),
        scratch_types=[
            pltpu.VMEM((_K // 4, 4 * _CH), jnp.int32),
            pltpu.VMEM((4 * _CH, 16), jnp.float32),
            pltpu.VMEM_SHARED((_NP, 16), jnp.float32),
            pltpu.SemaphoreType.DMA,
        ],
    )
    def deg_kernel(dst3, ones_h, zeros_h, out, dst_v, ones_v, acc_sh, dsem):
        c = lax.axis_index("c")
        s = lax.axis_index("s")
        wid = c * _NS + s
        pltpu.sync_copy(dst3.at[wid], dst_v)
        pltpu.sync_copy(ones_h, ones_v)
        pltpu.sync_copy(zeros_h.at[pl.ds(s * _RPS, _RPS)],
                        acc_sh.at[pl.ds(s * _RPS, _RPS)])
        plsc.subcore_barrier()

        @pl.loop(0, _K // 4)
        def _(j):
            pltpu.async_copy(ones_v, acc_sh.at[dst_v.at[j]], dsem, add=True)

        @pl.loop(0, _K // 4)
        def _(j):
            pltpu.make_async_copy(ones_v, acc_sh.at[dst_v.at[j]],
                                  dsem).wait()

        plsc.subcore_barrier()
        pltpu.sync_copy(acc_sh.at[pl.ds(s * _RPS, _RPS)],
                        out.at[pl.ds(c * _NP + s * _RPS, _RPS)])

    return deg_kernel


def _make_sc_agg():
    mesh = plsc.VectorSubcoreMesh(core_axis_name="c", subcore_axis_name="s")

    @functools.partial(
        pl.kernel,
        out_type=jax.ShapeDtypeStruct((_NC * _NP, _H), jnp.float32),
        mesh=mesh,
        compiler_params=pltpu.CompilerParams(use_tc_tiling_on_sc=False),
        scratch_types=[
            pltpu.VMEM((_K // 2, 2 * _CH), jnp.int32),
            pltpu.VMEM((_K // 2, 2 * _CH), jnp.int32),
            pltpu.VMEM((2 * _CH, _H), jnp.float32),
            pltpu.VMEM((2 * _CH, _H), jnp.float32),
            pltpu.VMEM((2 * _CH, _H), jnp.float32),
            pltpu.VMEM((2 * _CH, _H), jnp.float32),
            pltpu.VMEM_SHARED((_NP, _H), jnp.float32),
            pltpu.SemaphoreType.DMA,
            pltpu.SemaphoreType.DMA,
            pltpu.SemaphoreType.DMA,
            pltpu.SemaphoreType.DMA,
            pltpu.SemaphoreType.DMA,
            pltpu.SemaphoreType.DMA,
            pltpu.SemaphoreType.DMA,
            pltpu.SemaphoreType.DMA,
        ],
    )
    def agg_kernel(hs, src3, dst3, zeros_h, out, src_v, dst_v,
                   r0, r1, r2, r3, acc_sh, g0, g1, g2, g3, s0, s1, s2, s3):
        rows = (r0, r1, r2, r3)
        gsem = (g0, g1, g2, g3)
        ssem = (s0, s1, s2, s3)
        c = lax.axis_index("c")
        s = lax.axis_index("s")
        wid = c * _NS + s
        pltpu.sync_copy(src3.at[wid], src_v)
        pltpu.sync_copy(dst3.at[wid], dst_v)
        pltpu.sync_copy(zeros_h.at[pl.ds(s * _RPS, _RPS)],
                        acc_sh.at[pl.ds(s * _RPS, _RPS)])
        plsc.subcore_barrier()

        ng = _K // 2

        def gfire(g, k):
            pltpu.async_copy(hs.at[src_v.at[g]], rows[k], gsem[k])

        def gwait(g, k):
            pltpu.make_async_copy(hs.at[src_v.at[g]], rows[k],
                                  gsem[k]).wait()

        def sfire(g, k):
            pltpu.async_copy(rows[k], acc_sh.at[dst_v.at[g]], ssem[k],
                             add=True)

        def swait(g, k):
            pltpu.make_async_copy(rows[k], acc_sh.at[dst_v.at[g]],
                                  ssem[k]).wait()

        def step(g, k, first, last):
            gwait(g, k)
            sfire(g, k)
            if not last:
                if not first:
                    swait(g - 3, (k + 1) % 4)
                gfire(g + 1, (k + 1) % 4)

        gfire(0, 0)
        for k in range(4):
            step(k, k, True, False)

        @pl.loop(1, ng // 4 - 1)
        def _(h):
            for k in range(4):
                step(4 * h + k, k, False, False)

        for k in range(4):
            step(ng - 4 + k, k, False, k == 3)
        swait(ng - 3, 1)
        swait(ng - 2, 2)
        swait(ng - 1, 3)
        plsc.subcore_barrier()
        pltpu.sync_copy(acc_sh.at[pl.ds(s * _RPS, _RPS)],
                        out.at[pl.ds(c * _NP + s * _RPS, _RPS)])

    return agg_kernel


_sc_degree = _make_sc_degree()
_sc_agg = _make_sc_agg()


def _row_mask():
    rid = lax.broadcasted_iota(jnp.int32, (_BR, 1), 0) + pl.program_id(0) * _BR
    return (rid < _N).astype(jnp.float32)


def _prep_body(deg_ref, x_ref, w_ref, hs_ref, dinv_ref):
    d = deg_ref[0] + deg_ref[1]
    ones16 = jnp.ones((16, 1), jnp.float32)
    deg_col = lax.dot_general(d, ones16, (((1,), (0,)), ((), ())),
                              preferred_element_type=jnp.float32) * (1.0 / 16.0)
    dinv = lax.rsqrt(deg_col + 1.0)
    h = jnp.dot(x_ref[...], w_ref[...], preferred_element_type=jnp.float32)
    hs_ref[...] = h * (dinv * _row_mask())
    dinv_ref[...] = dinv


_tc_prep = pl.pallas_call(
    _prep_body,
    grid=(_NB,),
    in_specs=[
        pl.BlockSpec((2, _BR, 16), lambda i: (0, i, 0)),
        pl.BlockSpec((_BR, _D), lambda i: (i, 0)),
        pl.BlockSpec((_D, _H), lambda i: (0, 0)),
    ],
    out_specs=[
        pl.BlockSpec((_BR, _H), lambda i: (i, 0)),
        pl.BlockSpec((_BR, 1), lambda i: (i, 0)),
    ],
    out_shape=[
        jax.ShapeDtypeStruct((_NP, _H), jnp.float32),
        jax.ShapeDtypeStruct((_NP, 1), jnp.float32),
    ],
)


def _mid_body(acc_ref, hs_ref, dinv_ref, b_ref, w_ref, out_ref):
    dinv = dinv_ref[...]
    z = (acc_ref[0] + acc_ref[1] + hs_ref[...]) * dinv + b_ref[0:1, :]
    a = jnp.maximum(z, 0.0)
    out_ref[...] = jnp.dot(a, w_ref[...],
                           preferred_element_type=jnp.float32) * (
                               dinv * _row_mask())


_tc_mid = pl.pallas_call(
    _mid_body,
    grid=(_NB,),
    in_specs=[
        pl.BlockSpec((2, _BR, _H), lambda i: (0, i, 0)),
        pl.BlockSpec((_BR, _H), lambda i: (i, 0)),
        pl.BlockSpec((_BR, 1), lambda i: (i, 0)),
        pl.BlockSpec((8, _H), lambda i: (0, 0)),
        pl.BlockSpec((_H, _H), lambda i: (0, 0)),
    ],
    out_specs=pl.BlockSpec((_BR, _H), lambda i: (i, 0)),
    out_shape=jax.ShapeDtypeStruct((_NP, _H), jnp.float32),
)


def _final_body(acc_ref, hs_ref, dinv_ref, b_ref, batch_ref, out_ref,
                sums_ref, cnts_ref):
    i = pl.program_id(0)

    @pl.when(i == 0)
    def _():
        sums_ref[...] = jnp.zeros_like(sums_ref)
        cnts_ref[...] = jnp.zeros_like(cnts_ref)

    z = (acc_ref[0] + acc_ref[1] + hs_ref[...]) * dinv_ref[...] + b_ref[0:1, :]
    a = jnp.maximum(z, 0.0)
    gids = lax.broadcasted_iota(jnp.int32, (_BR, _G), 1).astype(jnp.float32)
    oh = (batch_ref[...] == gids).astype(jnp.float32)
    sums_ref[...] += lax.dot_general(oh, a, (((0,), (0,)), ((), ())),
                                     preferred_element_type=jnp.float32)
    cnts_ref[...] += lax.dot_general(oh, jnp.ones((_BR, 1), jnp.float32),
                                     (((0,), (0,)), ((), ())),
                                     preferred_element_type=jnp.float32)

    @pl.when(i == _NB - 1)
    def _():
        out_ref[...] = sums_ref[...] / jnp.maximum(cnts_ref[...], 1.0)


_tc_final = pl.pallas_call(
    _final_body,
    grid=(_NB,),
    in_specs=[
        pl.BlockSpec((2, _BR, _H), lambda i: (0, i, 0)),
        pl.BlockSpec((_BR, _H), lambda i: (i, 0)),
        pl.BlockSpec((_BR, 1), lambda i: (i, 0)),
        pl.BlockSpec((8, _H), lambda i: (0, 0)),
        pl.BlockSpec((_BR, 1), lambda i: (i, 0)),
    ],
    out_specs=pl.BlockSpec((_G, _H), lambda i: (0, 0)),
    out_shape=jax.ShapeDtypeStruct((_G, _H), jnp.float32),
    scratch_shapes=[
        pltpu.VMEM((_G, _H), jnp.float32),
        pltpu.VMEM((_G, 1), jnp.float32),
    ],
)


def kernel(x, edge_index, batch, W1, b1, gamma1, beta1, rm1, rv1,
           W2, b2, gamma2, beta2, rm2, rv2):
    f32 = jnp.float32
    s1 = gamma1 * lax.rsqrt(rv1 + _EPS)
    W1f = W1 * s1[None, :]
    b1f = jnp.tile((b1 * s1 + beta1 - rm1 * s1)[None, :], (8, 1))
    s2 = gamma2 * lax.rsqrt(rv2 + _EPS)
    W2f = W2 * s2[None, :]
    b2f = jnp.tile((b2 * s2 + beta2 - rm2 * s2)[None, :], (8, 1))

    x_p = jnp.pad(x, ((0, _NP - _N), (0, 0)))
    pad_e = _EP - _E
    pad_ar = jnp.arange(pad_e, dtype=jnp.int32)
    src_p = jnp.concatenate(
        [edge_index[0], _N + pad_ar % (_NP - _N)])
    dst_agg = jnp.concatenate([edge_index[1], pad_ar % _NP])
    dst_deg = jnp.concatenate([edge_index[1], _N + pad_ar % (_NP - _N)])
    src3 = src_p.reshape(_NW, _K // 2, 2 * _CH)
    dst3 = dst_agg.reshape(_NW, _K // 2, 2 * _CH)
    dstd3 = dst_deg.reshape(_NW, _K // 4, 4 * _CH)
    batch_p = jnp.concatenate(
        [batch, jnp.full((_NP - _N,), _G, jnp.int32)]).astype(f32)
    batch_p = batch_p.reshape(_NP, 1)

    ones16 = jnp.ones((4 * _CH, 16), f32)
    zeros16 = jnp.zeros((_NP, 16), f32)
    zeros64 = jnp.zeros((_NP, _H), f32)

    degacc = _sc_degree(dstd3, ones16, zeros16).reshape(_NC, _NP, 16)
    hs1, dinv = _tc_prep(degacc, x_p, W1f)
    acc1 = _sc_agg(hs1, src3, dst3, zeros64).reshape(_NC, _NP, _H)
    hs2 = _tc_mid(acc1, hs1, dinv, b1f, W2f)
    acc2 = _sc_agg(hs2, src3, dst3, zeros64).reshape(_NC, _NP, _H)
    return _tc_final(acc2, hs2, dinv, b2f, batch_p)


# speedup vs baseline: 1.0391x; 1.0391x over previous
import functools

import jax
import jax.numpy as jnp
from jax import lax
from jax.experimental import pallas as pl
from jax.experimental.pallas import tpu as pltpu
from jax.experimental.pallas import tpu_sc as plsc

_N = 10000
_E = 320000
_D = 128
_H = 64
_G = 64
_EPS = 1e-5

_NC = 2
_NS = 16
_NW = _NC * _NS
_CH = 128
_K = 80
_TILE_E = _K * _CH
_EP = _NW * _TILE_E
_NP = 10240
_RPS = _NP // _NS
_BR = 1024
_NB = _NP // _BR


def _make_sc_degree():
    mesh = plsc.VectorSubcoreMesh(core_axis_name="c", subcore_axis_name="s")

    @functools.partial(
        pl.kernel,
        out_type=jax.ShapeDtypeStruct((_NC * _NP, 16), jnp.float32),
        mesh=mesh,
        compiler_params=pltpu.CompilerParams(use_tc_tiling_on_sc=False),
        scratch_types=[
            pltpu.VMEM((_K // 4, 4 * _CH), jnp.int32),
            pltpu.VMEM((4 * _CH, 16), jnp.float32),
            pltpu.VMEM_SHARED((_NP, 16), jnp.float32),
        ],
    )
    def deg_kernel(dst3, ones_h, zeros_h, out, dst_v, ones_v, acc_sh):
        c = lax.axis_index("c")
        s = lax.axis_index("s")
        wid = c * _NS + s
        pltpu.sync_copy(dst3.at[wid], dst_v)
        pltpu.sync_copy(ones_h, ones_v)
        pltpu.sync_copy(zeros_h.at[pl.ds(s * _RPS, _RPS)],
                        acc_sh.at[pl.ds(s * _RPS, _RPS)])
        plsc.subcore_barrier()

        @pl.loop(0, _K // 4)
        def _(j):
            pltpu.sync_copy(ones_v, acc_sh.at[dst_v.at[j]], add=True)

        plsc.subcore_barrier()
        pltpu.sync_copy(acc_sh.at[pl.ds(s * _RPS, _RPS)],
                        out.at[pl.ds(c * _NP + s * _RPS, _RPS)])

    return deg_kernel


def _make_sc_agg():
    mesh = plsc.VectorSubcoreMesh(core_axis_name="c", subcore_axis_name="s")

    @functools.partial(
        pl.kernel,
        out_type=jax.ShapeDtypeStruct((_NC * _NP, _H), jnp.float32),
        mesh=mesh,
        compiler_params=pltpu.CompilerParams(use_tc_tiling_on_sc=False),
        scratch_types=[
            pltpu.VMEM((_K // 4, 4 * _CH), jnp.int32),
            pltpu.VMEM((_K // 4, 4 * _CH), jnp.int32),
            pltpu.VMEM((4 * _CH, _H), jnp.float32),
            pltpu.VMEM((4 * _CH, _H), jnp.float32),
            pltpu.VMEM_SHARED((_NP, _H), jnp.float32),
            pltpu.SemaphoreType.DMA,
            pltpu.SemaphoreType.DMA,
        ],
    )
    def agg_kernel(hs, src3, dst3, zeros_h, out, src_v, dst_v,
                   r0, r1, acc_sh, g0, g1):
        rows = (r0, r1)
        gsem = (g0, g1)
        c = lax.axis_index("c")
        s = lax.axis_index("s")
        wid = c * _NS + s
        pltpu.sync_copy(src3.at[wid], src_v)
        pltpu.sync_copy(dst3.at[wid], dst_v)
        pltpu.sync_copy(zeros_h.at[pl.ds(s * _RPS, _RPS)],
                        acc_sh.at[pl.ds(s * _RPS, _RPS)])
        plsc.subcore_barrier()

        ng = _K // 4

        def gfire(g, k):
            pltpu.async_copy(hs.at[src_v.at[g]], rows[k], gsem[k])

        def gwait(g, k):
            pltpu.make_async_copy(hs.at[src_v.at[g]], rows[k],
                                  gsem[k]).wait()

        def step(g, k, last):
            gwait(g, k)
            if not last:
                gfire(g + 1, 1 - k)
            pltpu.sync_copy(rows[k], acc_sh.at[dst_v.at[g]], add=True)

        gfire(0, 0)

        @pl.loop(0, ng // 2 - 1)
        def _(h):
            step(2 * h, 0, False)
            step(2 * h + 1, 1, False)

        step(ng - 2, 0, False)
        step(ng - 1, 1, True)
        plsc.subcore_barrier()
        pltpu.sync_copy(acc_sh.at[pl.ds(s * _RPS, _RPS)],
                        out.at[pl.ds(c * _NP + s * _RPS, _RPS)])

    return agg_kernel


_sc_degree = _make_sc_degree()
_sc_agg = _make_sc_agg()


def _row_mask():
    rid = lax.broadcasted_iota(jnp.int32, (_BR, 1), 0) + pl.program_id(0) * _BR
    return (rid < _N).astype(jnp.float32)


def _prep_body(deg_ref, x_ref, w_ref, hs_ref, dinv_ref):
    d = deg_ref[0] + deg_ref[1]
    ones16 = jnp.ones((16, 1), jnp.float32)
    deg_col = lax.dot_general(d, ones16, (((1,), (0,)), ((), ())),
                              preferred_element_type=jnp.float32) * (1.0 / 16.0)
    dinv = lax.rsqrt(deg_col + 1.0)
    h = jnp.dot(x_ref[...], w_ref[...], preferred_element_type=jnp.float32)
    hs_ref[...] = h * (dinv * _row_mask())
    dinv_ref[...] = dinv


_tc_prep = pl.pallas_call(
    _prep_body,
    grid=(_NB,),
    in_specs=[
        pl.BlockSpec((2, _BR, 16), lambda i: (0, i, 0)),
        pl.BlockSpec((_BR, _D), lambda i: (i, 0)),
        pl.BlockSpec((_D, _H), lambda i: (0, 0)),
    ],
    out_specs=[
        pl.BlockSpec((_BR, _H), lambda i: (i, 0)),
        pl.BlockSpec((_BR, 1), lambda i: (i, 0)),
    ],
    out_shape=[
        jax.ShapeDtypeStruct((_NP, _H), jnp.float32),
        jax.ShapeDtypeStruct((_NP, 1), jnp.float32),
    ],
)


def _mid_body(acc_ref, hs_ref, dinv_ref, b_ref, w_ref, out_ref):
    dinv = dinv_ref[...]
    z = (acc_ref[0] + acc_ref[1] + hs_ref[...]) * dinv + b_ref[0:1, :]
    a = jnp.maximum(z, 0.0)
    out_ref[...] = jnp.dot(a, w_ref[...],
                           preferred_element_type=jnp.float32) * (
                               dinv * _row_mask())


_tc_mid = pl.pallas_call(
    _mid_body,
    grid=(_NB,),
    in_specs=[
        pl.BlockSpec((2, _BR, _H), lambda i: (0, i, 0)),
        pl.BlockSpec((_BR, _H), lambda i: (i, 0)),
        pl.BlockSpec((_BR, 1), lambda i: (i, 0)),
        pl.BlockSpec((8, _H), lambda i: (0, 0)),
        pl.BlockSpec((_H, _H), lambda i: (0, 0)),
    ],
    out_specs=pl.BlockSpec((_BR, _H), lambda i: (i, 0)),
    out_shape=jax.ShapeDtypeStruct((_NP, _H), jnp.float32),
)


def _final_body(acc_ref, hs_ref, dinv_ref, b_ref, batch_ref, out_ref,
                sums_ref, cnts_ref):
    i = pl.program_id(0)

    @pl.when(i == 0)
    def _():
        sums_ref[...] = jnp.zeros_like(sums_ref)
        cnts_ref[...] = jnp.zeros_like(cnts_ref)

    z = (acc_ref[0] + acc_ref[1] + hs_ref[...]) * dinv_ref[...] + b_ref[0:1, :]
    a = jnp.maximum(z, 0.0)
    gids = lax.broadcasted_iota(jnp.int32, (_BR, _G), 1).astype(jnp.float32)
    oh = (batch_ref[...] == gids).astype(jnp.float32)
    sums_ref[...] += lax.dot_general(oh, a, (((0,), (0,)), ((), ())),
                                     preferred_element_type=jnp.float32)
    cnts_ref[...] += lax.dot_general(oh, jnp.ones((_BR, 1), jnp.float32),
                                     (((0,), (0,)), ((), ())),
                                     preferred_element_type=jnp.float32)

    @pl.when(i == _NB - 1)
    def _():
        out_ref[...] = sums_ref[...] / jnp.maximum(cnts_ref[...], 1.0)


_tc_final = pl.pallas_call(
    _final_body,
    grid=(_NB,),
    in_specs=[
        pl.BlockSpec((2, _BR, _H), lambda i: (0, i, 0)),
        pl.BlockSpec((_BR, _H), lambda i: (i, 0)),
        pl.BlockSpec((_BR, 1), lambda i: (i, 0)),
        pl.BlockSpec((8, _H), lambda i: (0, 0)),
        pl.BlockSpec((_BR, 1), lambda i: (i, 0)),
    ],
    out_specs=pl.BlockSpec((_G, _H), lambda i: (0, 0)),
    out_shape=jax.ShapeDtypeStruct((_G, _H), jnp.float32),
    scratch_shapes=[
        pltpu.VMEM((_G, _H), jnp.float32),
        pltpu.VMEM((_G, 1), jnp.float32),
    ],
)


def kernel(x, edge_index, batch, W1, b1, gamma1, beta1, rm1, rv1,
           W2, b2, gamma2, beta2, rm2, rv2):
    f32 = jnp.float32
    s1 = gamma1 * lax.rsqrt(rv1 + _EPS)
    W1f = W1 * s1[None, :]
    b1f = jnp.tile((b1 * s1 + beta1 - rm1 * s1)[None, :], (8, 1))
    s2 = gamma2 * lax.rsqrt(rv2 + _EPS)
    W2f = W2 * s2[None, :]
    b2f = jnp.tile((b2 * s2 + beta2 - rm2 * s2)[None, :], (8, 1))

    x_p = jnp.pad(x, ((0, _NP - _N), (0, 0)))
    pad_e = _EP - _E
    pad_ar = jnp.arange(pad_e, dtype=jnp.int32)
    src_p = jnp.concatenate(
        [edge_index[0], _N + pad_ar % (_NP - _N)])
    dst_agg = jnp.concatenate([edge_index[1], pad_ar % _NP])
    dst_deg = jnp.concatenate([edge_index[1], _N + pad_ar % (_NP - _N)])
    src3 = src_p.reshape(_NW, _K // 4, 4 * _CH)
    dst3 = dst_agg.reshape(_NW, _K // 4, 4 * _CH)
    dstd3 = dst_deg.reshape(_NW, _K // 4, 4 * _CH)
    batch_p = jnp.concatenate(
        [batch, jnp.full((_NP - _N,), _G, jnp.int32)]).astype(f32)
    batch_p = batch_p.reshape(_NP, 1)

    ones16 = jnp.ones((4 * _CH, 16), f32)
    zeros16 = jnp.zeros((_NP, 16), f32)
    zeros64 = jnp.zeros((_NP, _H), f32)

    degacc = _sc_degree(dstd3, ones16, zeros16).reshape(_NC, _NP, 16)
    hs1, dinv = _tc_prep(degacc, x_p, W1f)
    acc1 = _sc_agg(hs1, src3, dst3, zeros64).reshape(_NC, _NP, _H)
    hs2 = _tc_mid(acc1, hs1, dinv, b1f, W2f)
    acc2 = _sc_agg(hs2, src3, dst3, zeros64).reshape(_NC, _NP, _H)
    return _tc_final(acc2, hs2, dinv, b2f, batch_p)


# speedup vs baseline: 1.2511x; 1.2040x over previous
import functools

import jax
import jax.numpy as jnp
from jax import lax
from jax.experimental import pallas as pl
from jax.experimental.pallas import tpu as pltpu
from jax.experimental.pallas import tpu_sc as plsc

_N = 10000
_E = 320000
_D = 128
_H = 64
_G = 64
_EPS = 1e-5

_NC = 2
_NS = 16
_NW = _NC * _NS
_CH = 512
_NG = 20
_TILE_E = _NG * _CH
_EP = _NW * _TILE_E
_AR = 11136
_RPS = _N // _NS
_BR = 2000
_NB = _N // _BR


def _make_sc_degree():
    mesh = plsc.VectorSubcoreMesh(core_axis_name="c", subcore_axis_name="s")

    @functools.partial(
        pl.kernel,
        out_type=jax.ShapeDtypeStruct((_NC * _N, 128), jnp.float32),
        mesh=mesh,
        compiler_params=pltpu.CompilerParams(use_tc_tiling_on_sc=False),
        scratch_types=[
            pltpu.VMEM((_NG, _CH), jnp.int32),
            pltpu.VMEM((_CH, 16), jnp.float32),
            pltpu.VMEM_SHARED((_AR, 16), jnp.float32),
        ],
    )
    def deg_kernel(dst3, ones_h, zeros_h, out, dst_v, ones_v, acc_sh):
        c = lax.axis_index("c")
        s = lax.axis_index("s")
        wid = c * _NS + s
        pltpu.sync_copy(dst3.at[wid], dst_v)
        pltpu.sync_copy(ones_h, ones_v)
        pltpu.sync_copy(zeros_h.at[pl.ds(s * _RPS, _RPS)],
                        acc_sh.at[pl.ds(s * _RPS, _RPS)])
        plsc.subcore_barrier()

        @pl.loop(0, _NG)
        def _(j):
            pltpu.sync_copy(ones_v, acc_sh.at[dst_v.at[j]], add=True)

        plsc.subcore_barrier()
        pltpu.sync_copy(acc_sh.at[pl.ds(s * _RPS, _RPS)],
                        out.at[pl.ds(c * _N + s * _RPS, _RPS), pl.ds(0, 16)])

    return deg_kernel


def _make_sc_agg():
    mesh = plsc.VectorSubcoreMesh(core_axis_name="c", subcore_axis_name="s")

    @functools.partial(
        pl.kernel,
        out_type=jax.ShapeDtypeStruct((_NC * _N, 128), jnp.float32),
        mesh=mesh,
        compiler_params=pltpu.CompilerParams(use_tc_tiling_on_sc=False),
        scratch_types=[
            pltpu.VMEM((_NG, _CH), jnp.int32),
            pltpu.VMEM((_NG, _CH), jnp.int32),
            pltpu.VMEM((_CH, _H), jnp.float32),
            pltpu.VMEM((_CH, _H), jnp.float32),
            pltpu.VMEM_SHARED((_AR, _H), jnp.float32),
            pltpu.SemaphoreType.DMA,
            pltpu.SemaphoreType.DMA,
        ],
    )
    def agg_kernel(hs, src3, dst3, zeros_h, out, src_v, dst_v,
                   r0, r1, acc_sh, g0, g1):
        rows = (r0, r1)
        gsem = (g0, g1)
        c = lax.axis_index("c")
        s = lax.axis_index("s")
        wid = c * _NS + s
        pltpu.sync_copy(src3.at[wid], src_v)
        pltpu.sync_copy(dst3.at[wid], dst_v)
        pltpu.sync_copy(zeros_h.at[pl.ds(s * _RPS, _RPS)],
                        acc_sh.at[pl.ds(s * _RPS, _RPS)])
        plsc.subcore_barrier()

        def gfire(g, k):
            pltpu.async_copy(hs.at[src_v.at[g]], rows[k], gsem[k])

        def gwait(g, k):
            pltpu.make_async_copy(hs.at[src_v.at[g]], rows[k],
                                  gsem[k]).wait()

        def step(g, k, last):
            gwait(g, k)
            if not last:
                gfire(g + 1, 1 - k)
            pltpu.sync_copy(rows[k], acc_sh.at[dst_v.at[g]], add=True)

        gfire(0, 0)

        @pl.loop(0, _NG // 2 - 1)
        def _(h):
            step(2 * h, 0, False)
            step(2 * h + 1, 1, False)

        step(_NG - 2, 0, False)
        step(_NG - 1, 1, True)
        plsc.subcore_barrier()
        pltpu.sync_copy(acc_sh.at[pl.ds(s * _RPS, _RPS)],
                        out.at[pl.ds(c * _N + s * _RPS, _RPS), pl.ds(0, _H)])

    return agg_kernel


_sc_degree = _make_sc_degree()
_sc_agg = _make_sc_agg()


def _dinv_from_deg(deg_ref):
    d = deg_ref[0, :, 0:16] + deg_ref[1, :, 0:16]
    ones16 = jnp.ones((16, 1), jnp.float32)
    deg_col = lax.dot_general(d, ones16, (((1,), (0,)), ((), ())),
                              preferred_element_type=jnp.float32) * (1.0 / 16.0)
    return lax.rsqrt(deg_col + 1.0)


def _prep_body(deg_ref, x_ref, w_ref, hs_ref):
    dinv = _dinv_from_deg(deg_ref)
    h = jnp.dot(x_ref[...], w_ref[...], preferred_element_type=jnp.float32)
    hs_ref[:, 0:_H] = h * dinv
    hs_ref[:, _H:128] = jnp.zeros((_BR, 128 - _H), jnp.float32)


_tc_prep = pl.pallas_call(
    _prep_body,
    grid=(_NB,),
    in_specs=[
        pl.BlockSpec((2, _BR, 128), lambda i: (0, i, 0)),
        pl.BlockSpec((_BR, _D), lambda i: (i, 0)),
        pl.BlockSpec((_D, _H), lambda i: (0, 0)),
    ],
    out_specs=pl.BlockSpec((_BR, 128), lambda i: (i, 0)),
    out_shape=jax.ShapeDtypeStruct((_N, 128), jnp.float32),
)


def _mid_body(deg_ref, acc_ref, hs_ref, b_ref, w_ref, out_ref):
    dinv = _dinv_from_deg(deg_ref)
    z = (acc_ref[0, :, 0:_H] + acc_ref[1, :, 0:_H]
         + hs_ref[:, 0:_H]) * dinv + b_ref[0:1, :]
    a = jnp.maximum(z, 0.0)
    out_ref[:, 0:_H] = jnp.dot(a, w_ref[...],
                               preferred_element_type=jnp.float32) * dinv
    out_ref[:, _H:128] = jnp.zeros((_BR, 128 - _H), jnp.float32)


_tc_mid = pl.pallas_call(
    _mid_body,
    grid=(_NB,),
    in_specs=[
        pl.BlockSpec((2, _BR, 128), lambda i: (0, i, 0)),
        pl.BlockSpec((2, _BR, 128), lambda i: (0, i, 0)),
        pl.BlockSpec((_BR, 128), lambda i: (i, 0)),
        pl.BlockSpec((8, _H), lambda i: (0, 0)),
        pl.BlockSpec((_H, _H), lambda i: (0, 0)),
    ],
    out_specs=pl.BlockSpec((_BR, 128), lambda i: (i, 0)),
    out_shape=jax.ShapeDtypeStruct((_N, 128), jnp.float32),
)


def _final_body(deg_ref, acc_ref, hs_ref, b_ref, oneh_ref, out_ref,
                sums_ref, cnts_ref):
    i = pl.program_id(0)

    @pl.when(i == 0)
    def _():
        sums_ref[...] = jnp.zeros_like(sums_ref)
        cnts_ref[...] = jnp.zeros_like(cnts_ref)

    dinv = _dinv_from_deg(deg_ref)
    z = (acc_ref[0, :, 0:_H] + acc_ref[1, :, 0:_H]
         + hs_ref[:, 0:_H]) * dinv + b_ref[0:1, :]
    a = jnp.maximum(z, 0.0)
    oh = oneh_ref[...]
    sums_ref[...] += lax.dot_general(oh, a, (((0,), (0,)), ((), ())),
                                     preferred_element_type=jnp.float32)
    cnts_ref[...] += lax.dot_general(oh, jnp.ones((_BR, 1), jnp.float32),
                                     (((0,), (0,)), ((), ())),
                                     preferred_element_type=jnp.float32)

    @pl.when(i == _NB - 1)
    def _():
        out_ref[...] = sums_ref[...] / jnp.maximum(cnts_ref[...], 1.0)


_tc_final = pl.pallas_call(
    _final_body,
    grid=(_NB,),
    in_specs=[
        pl.BlockSpec((2, _BR, 128), lambda i: (0, i, 0)),
        pl.BlockSpec((2, _BR, 128), lambda i: (0, i, 0)),
        pl.BlockSpec((_BR, 128), lambda i: (i, 0)),
        pl.BlockSpec((8, _H), lambda i: (0, 0)),
        pl.BlockSpec((_BR, _G), lambda i: (i, 0)),
    ],
    out_specs=pl.BlockSpec((_G, _H), lambda i: (0, 0)),
    out_shape=jax.ShapeDtypeStruct((_G, _H), jnp.float32),
    scratch_shapes=[
        pltpu.VMEM((_G, _H), jnp.float32),
        pltpu.VMEM((_G, 1), jnp.float32),
    ],
)


def kernel(x, edge_index, batch, W1, b1, gamma1, beta1, rm1, rv1,
           W2, b2, gamma2, beta2, rm2, rv2):
    f32 = jnp.float32
    s1 = gamma1 * lax.rsqrt(rv1 + _EPS)
    W1f = W1 * s1[None, :]
    b1f = jnp.tile((b1 * s1 + beta1 - rm1 * s1)[None, :], (8, 1))
    s2 = gamma2 * lax.rsqrt(rv2 + _EPS)
    W2f = W2 * s2[None, :]
    b2f = jnp.tile((b2 * s2 + beta2 - rm2 * s2)[None, :], (8, 1))

    pad_e = _EP - _E
    pad_ar = jnp.arange(pad_e, dtype=jnp.int32)
    src_p = jnp.concatenate([edge_index[0], pad_ar % _N])
    dst_p = jnp.concatenate([edge_index[1], _N + pad_ar % (_AR - _N)])
    src3 = (src_p * 2).reshape(_NW, _NG, _CH)
    dst3 = dst_p.reshape(_NW, _NG, _CH)

    oneh = (batch[:, None] == jnp.arange(_G, dtype=batch.dtype)[None, :])
    oneh = oneh.astype(f32)

    ones16 = jnp.ones((_CH, 16), f32)
    zeros16 = jnp.zeros((_N, 16), f32)
    zeros64 = jnp.zeros((_N, _H), f32)

    degacc = _sc_degree(dst3, ones16, zeros16).reshape(_NC, _N, 128)
    hs1 = _tc_prep(degacc, x, W1f)
    acc1 = _sc_agg(hs1.reshape(2 * _N, _H), src3, dst3,
                   zeros64).reshape(_NC, _N, 128)
    hs2 = _tc_mid(degacc, acc1, hs1, b1f, W2f)
    acc2 = _sc_agg(hs2.reshape(2 * _N, _H), src3, dst3,
                   zeros64).reshape(_NC, _N, 128)
    return _tc_final(degacc, acc2, hs2, b2f, oneh)


# speedup vs baseline: 1.2644x; 1.0106x over previous
import functools

import jax
import jax.numpy as jnp
from jax import lax
from jax.experimental import pallas as pl
from jax.experimental.pallas import tpu as pltpu
from jax.experimental.pallas import tpu_sc as plsc

_N = 10000
_E = 320000
_D = 128
_H = 64
_G = 64
_EPS = 1e-5

_NC = 2
_NS = 16
_NW = _NC * _NS
_CH = 512
_NG = 20
_TILE_E = _NG * _CH
_EP = _NW * _TILE_E
_AR = 11136
_RPS = _N // _NS
_BR = 2000
_NB = _N // _BR


def _make_sc_degree():
    mesh = plsc.VectorSubcoreMesh(core_axis_name="c", subcore_axis_name="s")

    @functools.partial(
        pl.kernel,
        out_type=jax.ShapeDtypeStruct((_NC * _N, 128), jnp.float32),
        mesh=mesh,
        compiler_params=pltpu.CompilerParams(use_tc_tiling_on_sc=False),
        scratch_types=[
            pltpu.VMEM((_NG, _CH), jnp.int32),
            pltpu.VMEM((_CH, 16), jnp.float32),
            pltpu.VMEM_SHARED((_AR, 16), jnp.float32),
        ],
    )
    def deg_kernel(dst3, ones_h, zeros_h, out, dst_v, ones_v, acc_sh):
        c = lax.axis_index("c")
        s = lax.axis_index("s")
        wid = c * _NS + s
        pltpu.sync_copy(dst3.at[wid], dst_v)
        pltpu.sync_copy(ones_h, ones_v)
        pltpu.sync_copy(zeros_h.at[pl.ds(s * _RPS, _RPS)],
                        acc_sh.at[pl.ds(s * _RPS, _RPS)])
        plsc.subcore_barrier()

        @pl.loop(0, _NG)
        def _(j):
            pltpu.sync_copy(ones_v, acc_sh.at[dst_v.at[j]], add=True)

        plsc.subcore_barrier()
        pltpu.sync_copy(acc_sh.at[pl.ds(s * _RPS, _RPS)],
                        out.at[pl.ds(c * _N + s * _RPS, _RPS), pl.ds(0, 16)])

    return deg_kernel


def _make_sc_agg():
    mesh = plsc.VectorSubcoreMesh(core_axis_name="c", subcore_axis_name="s")

    @functools.partial(
        pl.kernel,
        out_type=jax.ShapeDtypeStruct((_NC * _N, 128), jnp.float32),
        mesh=mesh,
        compiler_params=pltpu.CompilerParams(use_tc_tiling_on_sc=False),
        scratch_types=[
            pltpu.VMEM((_NG, _CH), jnp.int32),
            pltpu.VMEM((_NG, _CH), jnp.int32),
            pltpu.VMEM((_CH, _H), jnp.float32),
            pltpu.VMEM((_CH, _H), jnp.float32),
            pltpu.VMEM_SHARED((_AR, _H), jnp.float32),
            pltpu.SemaphoreType.DMA,
            pltpu.SemaphoreType.DMA,
        ],
    )
    def agg_kernel(hs, src3, dst3, zeros_h, out, src_v, dst_v,
                   r0, r1, acc_sh, g0, g1):
        rows = (r0, r1)
        gsem = (g0, g1)
        c = lax.axis_index("c")
        s = lax.axis_index("s")
        wid = c * _NS + s
        pltpu.sync_copy(src3.at[wid], src_v)
        pltpu.sync_copy(dst3.at[wid], dst_v)
        pltpu.sync_copy(zeros_h.at[pl.ds(s * _RPS, _RPS)],
                        acc_sh.at[pl.ds(s * _RPS, _RPS)])
        plsc.subcore_barrier()

        def gfire(g, k):
            pltpu.async_copy(hs.at[src_v.at[g]], rows[k], gsem[k])

        def gwait(g, k):
            pltpu.make_async_copy(hs.at[src_v.at[g]], rows[k],
                                  gsem[k]).wait()

        def step(g, k, last):
            gwait(g, k)
            if not last:
                gfire(g + 1, 1 - k)
            pltpu.sync_copy(rows[k], acc_sh.at[dst_v.at[g]], add=True)

        gfire(0, 0)

        @pl.loop(0, _NG // 2 - 1)
        def _(h):
            step(2 * h, 0, False)
            step(2 * h + 1, 1, False)

        step(_NG - 2, 0, False)
        step(_NG - 1, 1, True)
        plsc.subcore_barrier()
        pltpu.sync_copy(acc_sh.at[pl.ds(s * _RPS, _RPS)],
                        out.at[pl.ds(c * _N + s * _RPS, _RPS), pl.ds(0, _H)])

    return agg_kernel


_sc_degree = _make_sc_degree()
_sc_agg = _make_sc_agg()


def _dinv_from_deg(deg_ref):
    d = deg_ref[0, :, 0:16] + deg_ref[1, :, 0:16]
    ones16 = jnp.ones((16, 1), jnp.float32)
    deg_col = lax.dot_general(d, ones16, (((1,), (0,)), ((), ())),
                              preferred_element_type=jnp.float32) * (1.0 / 16.0)
    return lax.rsqrt(deg_col + 1.0)


def _prep_body(deg_ref, x_ref, w_ref, hs_ref):
    dinv = _dinv_from_deg(deg_ref)
    h = jnp.dot(x_ref[...], w_ref[...], preferred_element_type=jnp.float32)
    hs_ref[:, 0:_H] = h * dinv
    hs_ref[:, _H:128] = jnp.zeros((_BR, 128 - _H), jnp.float32)


_tc_prep = pl.pallas_call(
    _prep_body,
    grid=(_NB,),
    in_specs=[
        pl.BlockSpec((2, _BR, 128), lambda i: (0, i, 0)),
        pl.BlockSpec((_BR, _D), lambda i: (i, 0)),
        pl.BlockSpec((_D, _H), lambda i: (0, 0)),
    ],
    out_specs=pl.BlockSpec((_BR, 128), lambda i: (i, 0)),
    out_shape=jax.ShapeDtypeStruct((_N, 128), jnp.float32),
)


def _mid_body(deg_ref, acc_ref, hs_ref, b_ref, w_ref, out_ref):
    dinv = _dinv_from_deg(deg_ref)
    z = (acc_ref[0, :, 0:_H] + acc_ref[1, :, 0:_H]
         + hs_ref[:, 0:_H]) * dinv + b_ref[0:1, :]
    a = jnp.maximum(z, 0.0)
    out_ref[:, 0:_H] = jnp.dot(a, w_ref[...],
                               preferred_element_type=jnp.float32) * dinv
    out_ref[:, _H:128] = jnp.zeros((_BR, 128 - _H), jnp.float32)


_tc_mid = pl.pallas_call(
    _mid_body,
    grid=(_NB,),
    in_specs=[
        pl.BlockSpec((2, _BR, 128), lambda i: (0, i, 0)),
        pl.BlockSpec((2, _BR, 128), lambda i: (0, i, 0)),
        pl.BlockSpec((_BR, 128), lambda i: (i, 0)),
        pl.BlockSpec((8, _H), lambda i: (0, 0)),
        pl.BlockSpec((_H, _H), lambda i: (0, 0)),
    ],
    out_specs=pl.BlockSpec((_BR, 128), lambda i: (i, 0)),
    out_shape=jax.ShapeDtypeStruct((_N, 128), jnp.float32),
)


def _final_body(deg_ref, acc_ref, hs_ref, b_ref, oneh_ref, out_ref,
                sums_ref, cnts_ref):
    i = pl.program_id(0)

    @pl.when(i == 0)
    def _():
        sums_ref[...] = jnp.zeros_like(sums_ref)
        cnts_ref[...] = jnp.zeros_like(cnts_ref)

    dinv = _dinv_from_deg(deg_ref)
    z = (acc_ref[0, :, 0:_H] + acc_ref[1, :, 0:_H]
         + hs_ref[:, 0:_H]) * dinv + b_ref[0:1, :]
    a = jnp.maximum(z, 0.0)
    bl = oneh_ref[0]
    oh_t = (lax.broadcasted_iota(jnp.int32, (_G, _BR), 0) == bl)
    oh_t = oh_t.astype(jnp.float32)
    sums_ref[...] += lax.dot_general(oh_t, a, (((1,), (0,)), ((), ())),
                                     preferred_element_type=jnp.float32)
    cnts_ref[...] += lax.dot_general(oh_t, jnp.ones((_BR, 1), jnp.float32),
                                     (((1,), (0,)), ((), ())),
                                     preferred_element_type=jnp.float32)

    @pl.when(i == _NB - 1)
    def _():
        out_ref[...] = sums_ref[...] / jnp.maximum(cnts_ref[...], 1.0)


_tc_final = pl.pallas_call(
    _final_body,
    grid=(_NB,),
    in_specs=[
        pl.BlockSpec((2, _BR, 128), lambda i: (0, i, 0)),
        pl.BlockSpec((2, _BR, 128), lambda i: (0, i, 0)),
        pl.BlockSpec((_BR, 128), lambda i: (i, 0)),
        pl.BlockSpec((8, _H), lambda i: (0, 0)),
        pl.BlockSpec((1, 1, _BR), lambda i: (i, 0, 0)),
    ],
    out_specs=pl.BlockSpec((_G, _H), lambda i: (0, 0)),
    out_shape=jax.ShapeDtypeStruct((_G, _H), jnp.float32),
    scratch_shapes=[
        pltpu.VMEM((_G, _H), jnp.float32),
        pltpu.VMEM((_G, 1), jnp.float32),
    ],
)


def kernel(x, edge_index, batch, W1, b1, gamma1, beta1, rm1, rv1,
           W2, b2, gamma2, beta2, rm2, rv2):
    f32 = jnp.float32
    s1 = gamma1 * lax.rsqrt(rv1 + _EPS)
    W1f = W1 * s1[None, :]
    b1f = jnp.tile((b1 * s1 + beta1 - rm1 * s1)[None, :], (8, 1))
    s2 = gamma2 * lax.rsqrt(rv2 + _EPS)
    W2f = W2 * s2[None, :]
    b2f = jnp.tile((b2 * s2 + beta2 - rm2 * s2)[None, :], (8, 1))

    pad_e = _EP - _E
    pad_ar = jnp.arange(pad_e, dtype=jnp.int32)
    src_p = jnp.concatenate([edge_index[0], pad_ar % _N])
    dst_p = jnp.concatenate([edge_index[1], _N + pad_ar % (_AR - _N)])
    src3 = (src_p * 2).reshape(_NW, _NG, _CH)
    dst3 = dst_p.reshape(_NW, _NG, _CH)

    batch2 = batch.reshape(_NB, 1, _BR)

    ones16 = jnp.ones((_CH, 16), f32)
    zeros16 = jnp.zeros((_N, 16), f32)
    zeros64 = jnp.zeros((_N, _H), f32)

    degacc = _sc_degree(dst3, ones16, zeros16).reshape(_NC, _N, 128)
    hs1 = _tc_prep(degacc, x, W1f)
    acc1 = _sc_agg(hs1.reshape(2 * _N, _H), src3, dst3,
                   zeros64).reshape(_NC, _N, 128)
    hs2 = _tc_mid(degacc, acc1, hs1, b1f, W2f)
    acc2 = _sc_agg(hs2.reshape(2 * _N, _H), src3, dst3,
                   zeros64).reshape(_NC, _N, 128)
    return _tc_final(degacc, acc2, hs2, b2f, batch2)
